# baseline (speedup 1.0000x reference)
import jax
import jax.numpy as jnp
from jax import lax
from jax.experimental import pallas as pl
from jax.experimental.pallas import tpu as pltpu
from jax.experimental.pallas import tpu_sc as plsc

_BM = 1024
_E = 64
_M = 16384
_NW = 32
_CHUNK = _M // _NW
_G = _CHUNK // 16


def _matmul_tile(x_ref, w_ref, lt_ref):
    logits = lax.dot_general(
        x_ref[...], w_ref[...], (((1,), (1,)), ((), ())),
        preferred_element_type=jnp.float32,
    )
    lt_ref[...] = logits.T


def _sc_top2(lt_hbm, valsT_hbm, idxT_hbm, buf, vals_buf, idx_buf):
    wid = lax.axis_index("s") * 2 + lax.axis_index("c")
    base = wid * _CHUNK
    pltpu.sync_copy(lt_hbm.at[:, pl.ds(base, _CHUNK)], buf)

    def group(g, _):
        col = pl.ds(g * 16, 16)

        def p1(e, m):
            return jnp.maximum(m, buf[e, col])

        m = lax.fori_loop(1, _E, p1, buf[0, col])

        def p2(e, z):
            ev = jnp.exp(buf[e, col] - m)
            buf[e, col] = ev
            return z + ev

        z = lax.fori_loop(0, _E, p2, jnp.zeros((16,), jnp.float32))

        def p3(e, carry):
            m1, m2, i1, i2 = carry
            s = buf[e, col] / z
            ei = jnp.full((16,), 0, jnp.int32) + e
            gt1 = s > m1
            gt2 = s > m2
            m2n = jnp.where(gt1, m1, jnp.where(gt2, s, m2))
            i2n = jnp.where(gt1, i1, jnp.where(gt2, ei, i2))
            m1n = jnp.where(gt1, s, m1)
            i1n = jnp.where(gt1, ei, i1)
            return (m1n, m2n, i1n, i2n)

        neg = jnp.full((16,), -1.0, jnp.float32)
        zi = jnp.zeros((16,), jnp.int32)
        m1, m2, i1, i2 = lax.fori_loop(0, _E, p3, (neg, neg, zi, zi))
        tot = m1 + m2
        vals_buf[0, col] = m1 / tot
        vals_buf[1, col] = m2 / tot
        idx_buf[0, col] = i1
        idx_buf[1, col] = i2
        return 0

    lax.fori_loop(0, _G, group, 0)
    pltpu.sync_copy(vals_buf, valsT_hbm.at[:, pl.ds(base, _CHUNK)])
    pltpu.sync_copy(idx_buf, idxT_hbm.at[:, pl.ds(base, _CHUNK)])


def kernel(X, W):
    B, T, K = X.shape
    M = B * T
    x2 = X.reshape(M, K)
    lt = pl.pallas_call(
        _matmul_tile,
        grid=(M // _BM,),
        in_specs=[
            pl.BlockSpec((_BM, K), lambda i: (i, 0)),
            pl.BlockSpec((_E, K), lambda i: (0, 0)),
        ],
        out_specs=pl.BlockSpec((_E, _BM), lambda i: (0, i)),
        out_shape=jax.ShapeDtypeStruct((_E, M), jnp.float32),
        compiler_params=pltpu.CompilerParams(
            dimension_semantics=("parallel",),
        ),
    )(x2, W)
    sc = pl.kernel(
        _sc_top2,
        mesh=plsc.VectorSubcoreMesh(core_axis_name="c", subcore_axis_name="s"),
        out_type=[
            jax.ShapeDtypeStruct((2, M), jnp.float32),
            jax.ShapeDtypeStruct((2, M), jnp.int32),
        ],
        scratch_types=[
            pltpu.VMEM((_E, _CHUNK), jnp.float32),
            pltpu.VMEM((2, _CHUNK), jnp.float32),
            pltpu.VMEM((2, _CHUNK), jnp.int32),
        ],
    )
    valsT, idxT = sc(lt)
    return valsT.T.reshape(B, T, 2), idxT.T.reshape(B, T, 2)


# speedup vs baseline: 1.4806x; 1.4806x over previous
import functools

import jax
import jax.numpy as jnp
from jax.experimental import pallas as pl
from jax.experimental.pallas import tpu as pltpu

_BM = 1024
_E = 64


def _router_tile(x_ref, w_ref, vals_ref, idx_ref):
    x = x_ref[...]
    w = w_ref[...]
    logits = jax.lax.dot_general(
        x, w, (((1,), (1,)), ((), ())), preferred_element_type=jnp.float32
    )
    e = jnp.exp(logits - jnp.max(logits, axis=1, keepdims=True))
    s = e / jnp.sum(e, axis=1, keepdims=True)
    col = jax.lax.broadcasted_iota(jnp.int32, s.shape, 1)
    m1 = jnp.max(s, axis=1, keepdims=True)
    i1 = jnp.min(jnp.where(s == m1, col, _E), axis=1, keepdims=True)
    masked = jnp.where(col == i1, -jnp.inf, s)
    m2 = jnp.max(masked, axis=1, keepdims=True)
    i2 = jnp.min(jnp.where(masked == m2, col, _E), axis=1, keepdims=True)
    tot = m1 + m2
    vals_ref[...] = jnp.concatenate([m1 / tot, m2 / tot], axis=1)
    idx_ref[...] = jnp.concatenate([i1, i2], axis=1)


@functools.partial(jax.jit, static_argnames=())
def kernel(X, W):
    B, T, K = X.shape
    M = B * T
    x2 = X.reshape(M, K)
    vals, idx = pl.pallas_call(
        _router_tile,
        grid=(M // _BM,),
        in_specs=[
            pl.BlockSpec((_BM, K), lambda i: (i, 0)),
            pl.BlockSpec((_E, K), lambda i: (0, 0)),
        ],
        out_specs=[
            pl.BlockSpec((_BM, 2), lambda i: (i, 0)),
            pl.BlockSpec((_BM, 2), lambda i: (i, 0)),
        ],
        out_shape=[
            jax.ShapeDtypeStruct((M, 2), jnp.float32),
            jax.ShapeDtypeStruct((M, 2), jnp.int32),
        ],
        compiler_params=pltpu.CompilerParams(
            dimension_semantics=("parallel",),
        ),
    )(x2, W)
    return vals.reshape(B, T, 2), idx.reshape(B, T, 2)
